# Initial kernel scaffold; baseline (speedup 1.0000x reference)
import jax
import jax.numpy as jnp
from jax import lax
from jax.experimental import pallas as pl
from jax.experimental.pallas import tpu as pltpu
from jax.experimental.pallas import tpu_sc as plsc

N = 10000
E = 160000
D = 256
H = 128
NS = 16
NC = 2

BC_TILES = NC * NS
BC_EPT = E // BC_TILES
BC_C = 40
BC_NCH = BC_EPT // BC_C
BPT = 640
NPAD = BPT * NS


def _bincount_body(src_hbm, out_hbm, idx_v, ones_v, fbuf, counts_sh):
    c = lax.axis_index("c")
    s = lax.axis_index("s")
    wid = c * NS + s
    pltpu.sync_copy(src_hbm.at[wid], idx_v)
    for i in range(3):
        ones_v[pl.ds(i * 16, 16)] = jnp.ones((16,), jnp.float32)

    def zbody(i, _):
        fbuf[pl.ds(i * 16, 16)] = jnp.zeros((16,), jnp.float32)
        return 0

    lax.fori_loop(0, BPT // 16, zbody, 0)
    pltpu.sync_copy(fbuf, counts_sh.at[pl.ds(s * BPT, BPT)])
    plsc.subcore_barrier()

    def sbody(j, _):
        pltpu.sync_copy(ones_v.at[pl.ds(0, BC_C)], counts_sh.at[idx_v.at[j]],
                        add=True)
        return 0

    lax.fori_loop(0, BC_NCH, sbody, 0)
    plsc.subcore_barrier()
    pltpu.sync_copy(counts_sh.at[pl.ds(s * BPT, BPT)], fbuf)
    pltpu.sync_copy(fbuf, out_hbm.at[c, pl.ds(s * BPT, BPT)])


_bincount = pl.kernel(
    _bincount_body,
    out_type=jax.ShapeDtypeStruct((NC, NPAD), jnp.float32),
    mesh=plsc.VectorSubcoreMesh(core_axis_name="c", subcore_axis_name="s"),
    scratch_types=[
        pltpu.VMEM((BC_NCH, BC_C), jnp.int32),
        pltpu.VMEM((48,), jnp.float32),
        pltpu.VMEM((BPT,), jnp.float32),
        pltpu.VMEM_SHARED((NPAD,), jnp.float32),
    ],
)

SP_EPT = E // NS
SP_C = 80
SP_NCH = SP_EPT // SP_C
ROWS_PT = N // NS
FB_ROWS = 125


def _spmm_body(r2_hbm, src_hbm, dst_hbm, out_hbm,
               src_v, dst_v, g0, g1, fbuf, acc_sh, sem0, sem1):
    c = lax.axis_index("c")
    s = lax.axis_index("s")
    pltpu.sync_copy(src_hbm.at[s], src_v)
    pltpu.sync_copy(dst_hbm.at[s], dst_v)

    def abody(j, _):
        sl = pl.ds(j * 16, 16)
        src_v[sl] = src_v[sl] * 2 + c
        return 0

    lax.fori_loop(0, SP_EPT // 16, abody, 0)

    def zbody(r, _):
        for k in range(H // 16):
            fbuf[r, pl.ds(k * 16, 16)] = jnp.zeros((16,), jnp.float32)
        return 0

    lax.fori_loop(0, FB_ROWS, zbody, 0)
    for q in range(ROWS_PT // FB_ROWS):
        pltpu.sync_copy(fbuf, acc_sh.at[pl.ds(s * ROWS_PT + q * FB_ROWS, FB_ROWS)])
    plsc.subcore_barrier()

    def fire(j, buf, sem):
        pltpu.async_copy(r2_hbm.at[src_v.at[pl.ds(j * SP_C, SP_C)]], buf, sem)

    def gwait(buf, sem):
        pltpu.make_async_copy(r2_hbm.at[src_v.at[pl.ds(0, SP_C)]], buf, sem).wait()

    def scat(j, buf):
        pltpu.sync_copy(buf, acc_sh.at[dst_v.at[j]], add=True)

    fire(0, g0, sem0)

    def mbody(i, _):
        gwait(g0, sem0)
        fire(2 * i + 1, g1, sem1)
        scat(2 * i, g0)
        gwait(g1, sem1)
        fire(2 * i + 2, g0, sem0)
        scat(2 * i + 1, g1)
        return 0

    lax.fori_loop(0, (SP_NCH - 1) // 2, mbody, 0)
    gwait(g0, sem0)
    scat(SP_NCH - 1, g0)
    plsc.subcore_barrier()
    for q in range(ROWS_PT // FB_ROWS):
        base = s * ROWS_PT + q * FB_ROWS
        pltpu.sync_copy(acc_sh.at[pl.ds(base, FB_ROWS)], fbuf)
        pltpu.sync_copy(fbuf, out_hbm.at[c, pl.ds(base, FB_ROWS)])


_spmm = pl.kernel(
    _spmm_body,
    out_type=jax.ShapeDtypeStruct((NC, N, H), jnp.float32),
    mesh=plsc.VectorSubcoreMesh(core_axis_name="c", subcore_axis_name="s"),
    scratch_types=[
        pltpu.VMEM((SP_EPT,), jnp.int32),
        pltpu.VMEM((SP_NCH, SP_C), jnp.int32),
        pltpu.VMEM((SP_C, H), jnp.float32),
        pltpu.VMEM((SP_C, H), jnp.float32),
        pltpu.VMEM((FB_ROWS, H), jnp.float32),
        pltpu.VMEM_SHARED((N, H), jnp.float32),
        pltpu.SemaphoreType.DMA,
        pltpu.SemaphoreType.DMA,
    ],
)

BM = 2000
NB = N // BM


def _mm_body(x_ref, w_ref, b_ref, h_ref):
    h_ref[...] = jnp.dot(x_ref[...], w_ref[...],
                         preferred_element_type=jnp.float32) + b_ref[...]


def _prep_body(h_ref, d0_ref, d1_ref, re_ref, r2_ref, p2_ref, nrm_ref):
    degs = d0_ref[...] + d1_ref[...] + 1.0
    norm = lax.rsqrt(degs)
    hv = h_ref[...]
    r2 = jnp.maximum(hv, 0.0) * norm[:, None]
    r2_ref[:, 0, :] = r2[:, :H]
    r2_ref[:, 1, :] = r2[:, H:]
    p2_ref[...] = jnp.maximum(hv + re_ref[...], 0.0) / degs[:, None]
    nrm_ref[...] = norm


def _comb_body(a_ref, n_ref, p_ref, o_ref):
    a = a_ref[...]
    n = n_ref[...]
    o_ref[:, :H] = a[0] * n[:, None] + p_ref[:, :H]
    o_ref[:, H:] = a[1] * n[:, None] + p_ref[:, H:]


@jax.jit
def kernel(x, edge_index, W, b, root_emb):
    src = edge_index[0]
    dst = edge_index[1]

    bc = _bincount(src.reshape(BC_TILES, BC_NCH, BC_C))
    d0 = bc[0, :N]
    d1 = bc[1, :N]

    h = pl.pallas_call(
        _mm_body,
        grid=(NB,),
        in_specs=[
            pl.BlockSpec((BM, D), lambda i: (i, 0)),
            pl.BlockSpec((D, D), lambda i: (0, 0)),
            pl.BlockSpec((1, D), lambda i: (0, 0)),
        ],
        out_specs=pl.BlockSpec((BM, D), lambda i: (i, 0)),
        out_shape=jax.ShapeDtypeStruct((N, D), jnp.float32),
    )(x, W, b.reshape(1, D))

    r2i, p2, nrm = pl.pallas_call(
        _prep_body,
        grid=(NB,),
        in_specs=[
            pl.BlockSpec((BM, D), lambda i: (i, 0)),
            pl.BlockSpec((BM,), lambda i: (i,)),
            pl.BlockSpec((BM,), lambda i: (i,)),
            pl.BlockSpec((1, D), lambda i: (0, 0)),
        ],
        out_specs=[
            pl.BlockSpec((BM, 2, H), lambda i: (i, 0, 0)),
            pl.BlockSpec((BM, D), lambda i: (i, 0)),
            pl.BlockSpec((BM,), lambda i: (i,)),
        ],
        out_shape=[
            jax.ShapeDtypeStruct((N, 2, H), jnp.float32),
            jax.ShapeDtypeStruct((N, D), jnp.float32),
            jax.ShapeDtypeStruct((N,), jnp.float32),
        ],
    )(h, d0, d1, root_emb)

    acc = _spmm(r2i.reshape(2 * N, H),
                src.reshape(NS, SP_EPT),
                dst.reshape(NS, SP_NCH, SP_C))

    out = pl.pallas_call(
        _comb_body,
        grid=(NB,),
        in_specs=[
            pl.BlockSpec((NC, BM, H), lambda i: (0, i, 0)),
            pl.BlockSpec((BM,), lambda i: (i,)),
            pl.BlockSpec((BM, D), lambda i: (i, 0)),
        ],
        out_specs=pl.BlockSpec((BM, D), lambda i: (i, 0)),
        out_shape=jax.ShapeDtypeStruct((N, D), jnp.float32),
    )(acc, nrm, p2)
    return out


# speedup vs baseline: 9.5891x; 9.5891x over previous
import jax
import jax.numpy as jnp
from jax import lax
from jax.experimental import pallas as pl
from jax.experimental.pallas import tpu as pltpu
from jax.experimental.pallas import tpu_sc as plsc

N = 10000
E = 160000
D = 256
H = 128
NS = 16
NC = 2

BC_TILES = NC * NS
BC_EPT = E // BC_TILES
BC_C = 40
BC_NCH = BC_EPT // BC_C
BPT = 640
NPAD = BPT * NS


def _bincount_body(src_hbm, out_hbm, idx_v, ones_v, fbuf, counts_sh):
    c = lax.axis_index("c")
    s = lax.axis_index("s")
    wid = c * NS + s
    pltpu.sync_copy(src_hbm.at[wid], idx_v)
    for i in range(3):
        ones_v[pl.ds(i * 16, 16)] = jnp.ones((16,), jnp.float32)

    def zbody(i, _):
        fbuf[pl.ds(i * 16, 16)] = jnp.zeros((16,), jnp.float32)
        return 0

    lax.fori_loop(0, BPT // 16, zbody, 0)
    pltpu.sync_copy(fbuf, counts_sh.at[pl.ds(s * BPT, BPT)])
    plsc.subcore_barrier()

    def sbody(j, _):
        pltpu.sync_copy(ones_v.at[pl.ds(0, BC_C)], counts_sh.at[idx_v.at[j]],
                        add=True)
        return 0

    lax.fori_loop(0, BC_NCH, sbody, 0)
    plsc.subcore_barrier()
    pltpu.sync_copy(counts_sh.at[pl.ds(s * BPT, BPT)], fbuf)
    pltpu.sync_copy(fbuf, out_hbm.at[c, s, 0])


_bincount = pl.kernel(
    _bincount_body,
    out_type=jax.ShapeDtypeStruct((NC, NS, 1, BPT), jnp.float32),
    mesh=plsc.VectorSubcoreMesh(core_axis_name="c", subcore_axis_name="s"),
    scratch_types=[
        pltpu.VMEM((BC_NCH, BC_C), jnp.int32),
        pltpu.VMEM((48,), jnp.float32),
        pltpu.VMEM((BPT,), jnp.float32),
        pltpu.VMEM_SHARED((NPAD,), jnp.float32),
    ],
)

SP_EPT = E // NS
SP_C = 80
SP_NCH = SP_EPT // SP_C
NROWS = NPAD
PH_ROWS = NROWS // 2
TRASH = PH_ROWS
ROWSA = 6144
ZR_PT = ROWSA // NS
FL_PT = PH_ROWS // NS
FB_ROWS = 128


def _spmm_body(r2_hbm, src_hbm, dst_hbm, out_hbm,
               src_v, dst_v, dst_a, g0, g1, fbuf, acc_sh, sem0, sem1):
    c = lax.axis_index("c")
    s = lax.axis_index("s")
    pltpu.sync_copy(src_hbm.at[pl.ds(s * SP_EPT, SP_EPT)], src_v)
    pltpu.sync_copy(dst_hbm.at[s], dst_v)

    def abody(j, _):
        sl = pl.ds(j * 16, 16)
        src_v[sl] = src_v[sl] * 2 + c
        return 0

    lax.fori_loop(0, SP_EPT // 16, abody, 0)

    def fire(j, buf, sem):
        pltpu.async_copy(r2_hbm.at[src_v.at[pl.ds(j * SP_C, SP_C)]], buf, sem)

    def gwait(buf, sem):
        pltpu.make_async_copy(r2_hbm.at[src_v.at[pl.ds(0, SP_C)]], buf, sem).wait()

    def scat(j, buf):
        pltpu.sync_copy(buf, acc_sh.at[dst_a.at[j]], add=True)

    for p in range(2):
        lo = p * PH_ROWS

        def cbody(r, _):
            for k in range(SP_C // 16):
                sl = pl.ds(k * 16, 16)
                t = dst_v[r, sl] - lo
                ok = (t >= 0) & (t < PH_ROWS)
                dst_a[r, sl] = jnp.where(ok, t, TRASH)
            return 0

        lax.fori_loop(0, SP_NCH, cbody, 0)

        def zbody(r, _):
            for k in range(H // 16):
                fbuf[r, pl.ds(k * 16, 16)] = jnp.zeros((16,), jnp.float32)
            return 0

        lax.fori_loop(0, FB_ROWS, zbody, 0)
        for q in range(ZR_PT // FB_ROWS):
            pltpu.sync_copy(fbuf,
                            acc_sh.at[pl.ds(s * ZR_PT + q * FB_ROWS, FB_ROWS)])
        plsc.subcore_barrier()

        fire(0, g0, sem0)

        def mbody(i, _):
            gwait(g0, sem0)
            fire(2 * i + 1, g1, sem1)
            scat(2 * i, g0)
            gwait(g1, sem1)
            fire(2 * i + 2, g0, sem0)
            scat(2 * i + 1, g1)
            return 0

        lax.fori_loop(0, (SP_NCH - 1) // 2, mbody, 0)
        gwait(g0, sem0)
        scat(SP_NCH - 1, g0)
        plsc.subcore_barrier()
        for q in range(FL_PT // 80):
            b = s * FL_PT + q * 80
            pltpu.sync_copy(acc_sh.at[pl.ds(b, 80)], fbuf.at[pl.ds(0, 80)])
            pltpu.sync_copy(fbuf.at[pl.ds(0, 80)], out_hbm.at[c, pl.ds(lo + b, 80)])
        plsc.subcore_barrier()


_spmm = pl.kernel(
    _spmm_body,
    out_type=jax.ShapeDtypeStruct((NC, NROWS, H), jnp.float32),
    mesh=plsc.VectorSubcoreMesh(core_axis_name="c", subcore_axis_name="s"),
    scratch_types=[
        pltpu.VMEM((SP_EPT,), jnp.int32),
        pltpu.VMEM((SP_NCH, SP_C), jnp.int32),
        pltpu.VMEM((SP_NCH, SP_C), jnp.int32),
        pltpu.VMEM((SP_C, H), jnp.float32),
        pltpu.VMEM((SP_C, H), jnp.float32),
        pltpu.VMEM((FB_ROWS, H), jnp.float32),
        pltpu.VMEM_SHARED((ROWSA, H), jnp.float32),
        pltpu.SemaphoreType.DMA,
        pltpu.SemaphoreType.DMA,
    ],
)

BM = 2000
NB = N // BM


def _mm_body(x_ref, w_ref, b_ref, h_ref):
    h_ref[...] = jnp.dot(x_ref[...], w_ref[...],
                         preferred_element_type=jnp.float32) + b_ref[...]


def _prep_body(h_ref, d0_ref, d1_ref, re_ref, r2_ref, p2_ref, nrm_ref):
    degs = d0_ref[...] + d1_ref[...] + 1.0
    norm = lax.rsqrt(degs)
    hv = h_ref[...]
    r2 = jnp.maximum(hv, 0.0) * norm
    r2_ref[:, 0, :] = r2[:, :H]
    r2_ref[:, 1, :] = r2[:, H:]
    p2_ref[...] = jnp.maximum(hv + re_ref[...], 0.0) / degs
    nrm_ref[...] = norm


def _comb_body(a_ref, n_ref, p_ref, o_ref):
    a = a_ref[...]
    n = n_ref[...]
    merged = jnp.concatenate([a[0], a[1]], axis=1)
    o_ref[...] = merged * n + p_ref[...]


@jax.jit
def kernel(x, edge_index, W, b, root_emb):
    src = edge_index[0]
    dst = edge_index[1]

    bc = _bincount(src.reshape(BC_TILES, BC_NCH, BC_C)).reshape(NC, NPAD)
    d0 = bc[0, :N].reshape(N, 1)
    d1 = bc[1, :N].reshape(N, 1)

    h = pl.pallas_call(
        _mm_body,
        grid=(NB,),
        in_specs=[
            pl.BlockSpec((BM, D), lambda i: (i, 0)),
            pl.BlockSpec((D, D), lambda i: (0, 0)),
            pl.BlockSpec((1, D), lambda i: (0, 0)),
        ],
        out_specs=pl.BlockSpec((BM, D), lambda i: (i, 0)),
        out_shape=jax.ShapeDtypeStruct((N, D), jnp.float32),
    )(x, W, b.reshape(1, D))

    r2i, p2, nrm = pl.pallas_call(
        _prep_body,
        grid=(NB,),
        in_specs=[
            pl.BlockSpec((BM, D), lambda i: (i, 0)),
            pl.BlockSpec((BM, 1), lambda i: (i, 0)),
            pl.BlockSpec((BM, 1), lambda i: (i, 0)),
            pl.BlockSpec((1, D), lambda i: (0, 0)),
        ],
        out_specs=[
            pl.BlockSpec((BM, NC, H), lambda i: (i, 0, 0)),
            pl.BlockSpec((BM, D), lambda i: (i, 0)),
            pl.BlockSpec((BM, 1), lambda i: (i, 0)),
        ],
        out_shape=[
            jax.ShapeDtypeStruct((N, NC, H), jnp.float32),
            jax.ShapeDtypeStruct((N, D), jnp.float32),
            jax.ShapeDtypeStruct((N, 1), jnp.float32),
        ],
    )(h, d0, d1, root_emb)

    acc = _spmm(r2i.reshape(NC * N, H),
                src,
                dst.reshape(NS, SP_NCH, SP_C))

    out = pl.pallas_call(
        _comb_body,
        grid=(NB,),
        in_specs=[
            pl.BlockSpec((NC, BM, H), lambda i: (0, i, 0)),
            pl.BlockSpec((BM, 1), lambda i: (i, 0)),
            pl.BlockSpec((BM, D), lambda i: (i, 0)),
        ],
        out_specs=pl.BlockSpec((BM, D), lambda i: (i, 0)),
        out_shape=jax.ShapeDtypeStruct((N, D), jnp.float32),
    )(acc, nrm, p2)
    return out
